# Initial kernel scaffold; baseline (speedup 1.0000x reference)
import jax
import jax.numpy as jnp
from jax.experimental import pallas as pl


def kernel(image, feat, rpn_w, rpn_b, cls_w, cls_b, rgs_w, rgs_b):
    raise NotImplementedError("write your pallas kernel here")


# speedup vs baseline: 15.2784x; 15.2784x over previous
import functools
import math

import jax
import jax.numpy as jnp
from jax.experimental import pallas as pl
from jax.experimental.pallas import tpu as pltpu

_SCALES = [128.0, 256.0, 512.0]
_RATIOS = [0.5, 1.0, 2.0]
_K = 9
_TOPK = 2000
_MIN_SIZE = 16.0
_NMS_THRESH = 0.7
_BBOX_CLIP = math.log(1000.0 / 16)

_N = 5184
_PAD_N = 8192
_NBLK = 41
_ACT = _NBLK * 128
_OUT_PAD = 2048

_HIGHEST = jax.lax.Precision.HIGHEST


def _anchors(image_shape, feat_shape):
    grid_h, grid_w = feat_shape[-2], feat_shape[-1]
    stride_h = image_shape[-2] // grid_h
    stride_w = image_shape[-1] // grid_w
    scales = jnp.asarray(_SCALES, jnp.float32)
    ratios = jnp.asarray(_RATIOS, jnp.float32)
    h_ratio = jnp.sqrt(ratios)
    w_ratio = 1.0 / h_ratio
    ws = (w_ratio[:, None] * scales[None, :]).reshape(-1)
    hs = (h_ratio[:, None] * scales[None, :]).reshape(-1)
    base = jnp.round(jnp.stack([-ws, -hs, ws, hs], axis=1) / 2.0)
    sx = jnp.arange(grid_w, dtype=jnp.float32) * stride_w
    sy = jnp.arange(grid_h, dtype=jnp.float32) * stride_h
    SY, SX = jnp.meshgrid(sy, sx, indexing='ij')
    shifts = jnp.stack([SX.reshape(-1), SY.reshape(-1),
                        SX.reshape(-1), SY.reshape(-1)], axis=1)
    return (shifts[:, None, :] + base[None, :, :]).reshape(-1, 4)


def _trunk_kernel(x_ref, w_ref, clsw_ref, rgsw_ref, rpnb_ref, clsb_ref,
                  rgsb_ref, cls_out, rgs_out):
    acc = jnp.dot(x_ref[...], w_ref[...], preferred_element_type=jnp.float32)
    feat = jnp.maximum(acc + rpnb_ref[...], 0.0)
    cls_out[...] = jnp.dot(feat, clsw_ref[...],
                           preferred_element_type=jnp.float32) + clsb_ref[...]
    rgs_out[...] = jnp.dot(feat, rgsw_ref[...],
                           preferred_element_type=jnp.float32) + rgsb_ref[...]


def _run_trunk(xcat, wcat, cls_w2, rgs_w2, rpn_b, cls_b, rgs_b):
    return pl.pallas_call(
        _trunk_kernel,
        in_specs=[
            pl.BlockSpec((576, 4608), lambda: (0, 0)),
            pl.BlockSpec((4608, 512), lambda: (0, 0)),
            pl.BlockSpec((512, _K), lambda: (0, 0)),
            pl.BlockSpec((512, 4 * _K), lambda: (0, 0)),
            pl.BlockSpec((1, 512), lambda: (0, 0)),
            pl.BlockSpec((1, _K), lambda: (0, 0)),
            pl.BlockSpec((1, 4 * _K), lambda: (0, 0)),
        ],
        out_specs=[
            pl.BlockSpec((576, _K), lambda: (0, 0)),
            pl.BlockSpec((576, 4 * _K), lambda: (0, 0)),
        ],
        out_shape=[
            jax.ShapeDtypeStruct((576, _K), jnp.float32),
            jax.ShapeDtypeStruct((576, 4 * _K), jnp.float32),
        ],
    )(xcat, wcat, cls_w2, rgs_w2, rpn_b, cls_b, rgs_b)


def _iota2(shape, dim):
    return jax.lax.broadcasted_iota(jnp.int32, shape, dim)


def _colify(v, eye):
    return jnp.sum(v * eye, axis=1, keepdims=True)


def _rowify(c, eye):
    return jnp.sum(c * eye, axis=0, keepdims=True)


def _filter_kernel(logit_ref, dl_ref, an_ref, out_ref, s_scr, *, img_h, img_w):
    f32 = jnp.float32
    eye = (_iota2((128, 128), 0) == _iota2((128, 128), 1)).astype(f32)
    gi = _iota2((64, 128), 0) * 128 + _iota2((64, 128), 1)

    ax1, ay1, ax2, ay2 = an_ref[0], an_ref[1], an_ref[2], an_ref[3]
    dx, dy, dw, dh = dl_ref[0], dl_ref[1], dl_ref[2], dl_ref[3]
    aw = ax2 - ax1
    ah = ay2 - ay1
    acx = ax1 + 0.5 * aw
    acy = ay1 + 0.5 * ah
    dw = jnp.minimum(dw, _BBOX_CLIP)
    dh = jnp.minimum(dh, _BBOX_CLIP)
    pcx = dx * aw + acx
    pcy = dy * ah + acy
    pw = jnp.exp(dw) * aw
    ph = jnp.exp(dh) * ah
    x1 = jnp.clip(pcx - pw / 2, 0.0, img_w)
    y1 = jnp.clip(pcy - ph / 2, 0.0, img_h)
    x2 = jnp.clip(pcx + pw / 2, 0.0, img_w)
    y2 = jnp.clip(pcy + ph / 2, 0.0, img_h)
    valid = ((x2 - x1) >= _MIN_SIZE) & ((y2 - y1) >= _MIN_SIZE)

    logit = logit_ref[...]
    sig = 1.0 / (1.0 + jnp.exp(-logit))
    score = jnp.where(gi < _N, jnp.where(valid, sig, -1.0), -2.0)

    small_p = {d: ((_iota2((128, 128), 0) ^ d) == _iota2((128, 128), 1)
                   ).astype(f32) for d in (1, 2, 4, 8, 16, 32, 64)}
    big_q = {dr: ((_iota2((64, 64), 0) ^ dr) == _iota2((64, 64), 1)
                  ).astype(f32) for dr in (1, 2, 4, 8, 16, 32)}

    arrs = [score, gi.astype(f32), x1, y1, x2, y2]
    for kk in [2 << j for j in range(13)]:
        d = kk // 2
        while d >= 1:
            if d >= 128:
                q = big_q[d // 128]
                parts = [jnp.dot(q, a, preferred_element_type=f32,
                                 precision=_HIGHEST) for a in arrs]
            else:
                p = small_p[d]
                parts = [jnp.dot(a, p, preferred_element_type=f32,
                                 precision=_HIGHEST) for a in arrs]
            b1 = (gi & kk) == 0
            b2 = (gi & d) == 0
            want_max = jnp.logical_not(jnp.logical_xor(b1, b2))
            s, sp = arrs[0], parts[0]
            i0, ip = arrs[1], parts[1]
            ties = sp == s
            win_hi = (sp > s) | (ties & (ip < i0))
            win_lo = (sp < s) | (ties & (ip > i0))
            win = (want_max & win_hi) | (jnp.logical_not(want_max) & win_lo)
            arrs = [jnp.where(win, ap, a) for a, ap in zip(arrs, parts)]
            d //= 2
    score, _, x1, y1, x2, y2 = arrs

    def col_cat(a):
        return jnp.concatenate(
            [_colify(a[r:r + 1, :], eye) for r in range(_NBLK)], axis=0)

    x1c, y1c, x2c, y2c, sc_c = (col_cat(a) for a in (x1, y1, x2, y2, score))
    area_row = (x2 - x1) * (y2 - y1)
    area_col = (x2c - x1c) * (y2c - y1c)

    keepcol = (sc_c > 0.0).astype(f32)

    lane_iota = _iota2((1, 128), 1)
    up_tri = (_iota2((128, 128), 1) > _iota2((128, 128), 0)).astype(f32)

    for bi in range(_NBLK):
        lo = bi * 128
        rx1, ry1, rx2, ry2 = (a[bi:bi + 1, :] for a in (x1, y1, x2, y2))
        cx1, cy1, cx2, cy2 = (c[lo:lo + 128] for c in (x1c, y1c, x2c, y2c))
        keep_row = _rowify(keepcol[lo:lo + 128], eye)

        xl = jnp.maximum(cx1, rx1)
        yt = jnp.maximum(cy1, ry1)
        xr = jnp.minimum(cx2, rx2)
        yb = jnp.minimum(cy2, ry2)
        inter = jnp.maximum(xr - xl, 0.0) * jnp.maximum(yb - yt, 0.0)
        iou = inter / (area_col[lo:lo + 128] + area_row[bi:bi + 1, :] - inter)
        s_scr[...] = jnp.where(iou > _NMS_THRESH, up_tri, 0.0)

        def body(p, kr):
            srow = s_scr[pl.ds(p, 1), :]
            kp = jnp.sum(jnp.where(lane_iota == p, kr, 0.0))
            return kr * (1.0 - srow * kp)

        keep_row = jax.lax.fori_loop(0, 128, body, keep_row)
        pieces = ([keepcol[:lo]] if lo else []) + [_colify(keep_row, eye)]

        if lo + 128 < _ACT:
            qx1 = x1c[lo + 128:]
            qy1 = y1c[lo + 128:]
            qx2 = x2c[lo + 128:]
            qy2 = y2c[lo + 128:]
            xl = jnp.maximum(qx1, rx1)
            yt = jnp.maximum(qy1, ry1)
            xr = jnp.minimum(qx2, rx2)
            yb = jnp.minimum(qy2, ry2)
            inter = jnp.maximum(xr - xl, 0.0) * jnp.maximum(yb - yt, 0.0)
            iou = inter / (area_col[lo + 128:] + area_row[bi:bi + 1, :] - inter)
            hit = jnp.where(iou > _NMS_THRESH, keep_row, 0.0)
            sup = jnp.sum(hit, axis=1, keepdims=True)
            pieces.append(keepcol[lo + 128:] * jnp.where(sup > 0.0, 0.0, 1.0))
        keepcol = jnp.concatenate(pieces, axis=0)

    keeprows = jnp.concatenate(
        [_rowify(keepcol[r * 128:(r + 1) * 128], eye) for r in range(_NBLK)],
        axis=0)
    triu = (_iota2((128, 128), 0) <= _iota2((128, 128), 1)).astype(f32)
    lane_cum = jnp.dot(keeprows, triu, preferred_element_type=f32,
                       precision=_HIGHEST)
    row_tot = lane_cum[:, 127:128]
    strict_lo = (_iota2((_NBLK, _NBLK), 0) > _iota2((_NBLK, _NBLK), 1)
                 ).astype(f32)
    row_ex = jnp.dot(strict_lo, row_tot, preferred_element_type=f32,
                     precision=_HIGHEST)
    posi = (lane_cum + row_ex - 1.0).astype(jnp.int32)

    out_iota = _iota2((_OUT_PAD, 128), 0)
    acc = jnp.zeros((_OUT_PAD, 8), f32)
    zeros3 = jnp.zeros((128, 3), f32)
    for bi in range(_NBLK):
        lo = bi * 128
        hot = jnp.where(out_iota == posi[bi:bi + 1, :],
                        keeprows[bi:bi + 1, :], 0.0)
        vals = jnp.concatenate(
            [x1c[lo:lo + 128], y1c[lo:lo + 128], x2c[lo:lo + 128],
             y2c[lo:lo + 128], sc_c[lo:lo + 128], zeros3], axis=1)
        acc = acc + jnp.dot(hot, vals, preferred_element_type=f32,
                            precision=_HIGHEST)
    out_ref[...] = acc


def _run_filter(logit, deltas, anchors, img_h, img_w):
    return pl.pallas_call(
        functools.partial(_filter_kernel, img_h=img_h, img_w=img_w),
        in_specs=[
            pl.BlockSpec((64, 128), lambda: (0, 0)),
            pl.BlockSpec((4, 64, 128), lambda: (0, 0, 0)),
            pl.BlockSpec((4, 64, 128), lambda: (0, 0, 0)),
        ],
        out_specs=pl.BlockSpec((_OUT_PAD, 8), lambda: (0, 0)),
        out_shape=jax.ShapeDtypeStruct((_OUT_PAD, 8), jnp.float32),
        scratch_shapes=[pltpu.VMEM((128, 128), jnp.float32)],
    )(logit, deltas, anchors)


def kernel(image, feat, rpn_w, rpn_b, cls_w, cls_b, rgs_w, rgs_b):
    f32 = jnp.float32
    xt = jnp.pad(jnp.transpose(feat[0], (1, 2, 0)), ((1, 1), (1, 1), (0, 0)))
    xcat = jnp.concatenate([xt[dy:dy + 24, dx:dx + 24, :].reshape(576, 512)
                            for dy in range(3) for dx in range(3)], axis=1)
    wcat = jnp.transpose(rpn_w, (2, 3, 1, 0)).reshape(4608, 512)
    cls_w2 = jnp.transpose(cls_w[:, :, 0, 0], (1, 0))
    rgs_w2 = jnp.transpose(rgs_w[:, :, 0, 0], (1, 0))

    cls_out, rgs_out = _run_trunk(
        xcat, wcat, cls_w2, rgs_w2, rpn_b.reshape(1, 512),
        cls_b.reshape(1, _K), rgs_b.reshape(1, 4 * _K))

    def to_grid(v):
        return jnp.pad(v, (0, _PAD_N - _N)).reshape(64, 128)

    logit = to_grid(cls_out.reshape(_N))
    d4 = rgs_out.reshape(_N, 4)
    deltas = jnp.stack([to_grid(d4[:, i]) for i in range(4)])
    anc = _anchors(image.shape, feat.shape)
    anchors = jnp.stack([to_grid(anc[:, i]) for i in range(4)])

    out = _run_filter(logit, deltas, anchors,
                      float(image.shape[-2]), float(image.shape[-1]))
    return out[:_TOPK, 0:4], out[:_TOPK, 4]


# speedup vs baseline: 64.6655x; 4.2325x over previous
import functools
import math

import jax
import jax.numpy as jnp
from jax.experimental import pallas as pl
from jax.experimental.pallas import tpu as pltpu

_SCALES = [128.0, 256.0, 512.0]
_RATIOS = [0.5, 1.0, 2.0]
_K = 9
_TOPK = 2000
_MIN_SIZE = 16.0
_NMS_THRESH = 0.7
_BBOX_CLIP = math.log(1000.0 / 16)

_N = 5184
_PAD_N = 8192
_NBLK = 41
_ACT = _NBLK * 128
_OUT_PAD = 2048

_HIGHEST = jax.lax.Precision.HIGHEST


def _anchors(image_shape, feat_shape):
    grid_h, grid_w = feat_shape[-2], feat_shape[-1]
    stride_h = image_shape[-2] // grid_h
    stride_w = image_shape[-1] // grid_w
    scales = jnp.asarray(_SCALES, jnp.float32)
    ratios = jnp.asarray(_RATIOS, jnp.float32)
    h_ratio = jnp.sqrt(ratios)
    w_ratio = 1.0 / h_ratio
    ws = (w_ratio[:, None] * scales[None, :]).reshape(-1)
    hs = (h_ratio[:, None] * scales[None, :]).reshape(-1)
    base = jnp.round(jnp.stack([-ws, -hs, ws, hs], axis=1) / 2.0)
    sx = jnp.arange(grid_w, dtype=jnp.float32) * stride_w
    sy = jnp.arange(grid_h, dtype=jnp.float32) * stride_h
    SY, SX = jnp.meshgrid(sy, sx, indexing='ij')
    shifts = jnp.stack([SX.reshape(-1), SY.reshape(-1),
                        SX.reshape(-1), SY.reshape(-1)], axis=1)
    return (shifts[:, None, :] + base[None, :, :]).reshape(-1, 4)


def _trunk_kernel(x_ref, w_ref, clsw_ref, rgsw_ref, rpnb_ref, clsb_ref,
                  rgsb_ref, cls_out, rgs_out):
    acc = jnp.dot(x_ref[...], w_ref[...], preferred_element_type=jnp.float32)
    feat = jnp.maximum(acc + rpnb_ref[...], 0.0)
    cls_out[...] = jnp.dot(feat, clsw_ref[...],
                           preferred_element_type=jnp.float32) + clsb_ref[...]
    rgs_out[...] = jnp.dot(feat, rgsw_ref[...],
                           preferred_element_type=jnp.float32) + rgsb_ref[...]


def _run_trunk(xcat, wcat, cls_w2, rgs_w2, rpn_b, cls_b, rgs_b):
    return pl.pallas_call(
        _trunk_kernel,
        in_specs=[
            pl.BlockSpec((576, 4608), lambda: (0, 0)),
            pl.BlockSpec((4608, 512), lambda: (0, 0)),
            pl.BlockSpec((512, _K), lambda: (0, 0)),
            pl.BlockSpec((512, 4 * _K), lambda: (0, 0)),
            pl.BlockSpec((1, 512), lambda: (0, 0)),
            pl.BlockSpec((1, _K), lambda: (0, 0)),
            pl.BlockSpec((1, 4 * _K), lambda: (0, 0)),
        ],
        out_specs=[
            pl.BlockSpec((576, _K), lambda: (0, 0)),
            pl.BlockSpec((576, 4 * _K), lambda: (0, 0)),
        ],
        out_shape=[
            jax.ShapeDtypeStruct((576, _K), jnp.float32),
            jax.ShapeDtypeStruct((576, 4 * _K), jnp.float32),
        ],
    )(xcat, wcat, cls_w2, rgs_w2, rpn_b, cls_b, rgs_b)


def _iota2(shape, dim):
    return jax.lax.broadcasted_iota(jnp.int32, shape, dim)


def _colify(v, eye):
    return jnp.sum(v * eye, axis=1, keepdims=True)


def _rowify(c, eye):
    return jnp.sum(c * eye, axis=0, keepdims=True)


def _filter_kernel(logit_ref, dl_ref, an_ref, out_ref, *, img_h, img_w):
    f32 = jnp.float32
    eye = (_iota2((128, 128), 0) == _iota2((128, 128), 1)).astype(f32)
    gi = _iota2((64, 128), 0) * 128 + _iota2((64, 128), 1)

    ax1, ay1, ax2, ay2 = an_ref[0], an_ref[1], an_ref[2], an_ref[3]
    dx, dy, dw, dh = dl_ref[0], dl_ref[1], dl_ref[2], dl_ref[3]
    aw = ax2 - ax1
    ah = ay2 - ay1
    acx = ax1 + 0.5 * aw
    acy = ay1 + 0.5 * ah
    dw = jnp.minimum(dw, _BBOX_CLIP)
    dh = jnp.minimum(dh, _BBOX_CLIP)
    pcx = dx * aw + acx
    pcy = dy * ah + acy
    pw = jnp.exp(dw) * aw
    ph = jnp.exp(dh) * ah
    x1 = jnp.clip(pcx - pw / 2, 0.0, img_w)
    y1 = jnp.clip(pcy - ph / 2, 0.0, img_h)
    x2 = jnp.clip(pcx + pw / 2, 0.0, img_w)
    y2 = jnp.clip(pcy + ph / 2, 0.0, img_h)
    valid = ((x2 - x1) >= _MIN_SIZE) & ((y2 - y1) >= _MIN_SIZE)

    logit = logit_ref[...]
    sig = 1.0 / (1.0 + jnp.exp(-logit))
    score = jnp.where(gi < _N, jnp.where(valid, sig, -1.0), -2.0)

    small_p = {d: ((_iota2((128, 128), 0) ^ d) == _iota2((128, 128), 1)
                   ).astype(f32) for d in (1, 2, 4, 8, 16, 32, 64)}
    big_q = {dr: ((_iota2((64, 64), 0) ^ dr) == _iota2((64, 64), 1)
                  ).astype(f32) for dr in (1, 2, 4, 8, 16, 32)}

    arrs = [score, gi.astype(f32), x1, y1, x2, y2]
    for kk in [2 << j for j in range(13)]:
        d = kk // 2
        while d >= 1:
            if d >= 128:
                q = big_q[d // 128]
                parts = [jnp.dot(q, a, preferred_element_type=f32,
                                 precision=_HIGHEST) for a in arrs]
            else:
                p = small_p[d]
                parts = [jnp.dot(a, p, preferred_element_type=f32,
                                 precision=_HIGHEST) for a in arrs]
            b1 = (gi & kk) == 0
            b2 = (gi & d) == 0
            want_max = jnp.logical_not(jnp.logical_xor(b1, b2))
            s, sp = arrs[0], parts[0]
            i0, ip = arrs[1], parts[1]
            ties = sp == s
            win_hi = (sp > s) | (ties & (ip < i0))
            win_lo = (sp < s) | (ties & (ip > i0))
            win = (want_max & win_hi) | (jnp.logical_not(want_max) & win_lo)
            arrs = [jnp.where(win, ap, a) for a, ap in zip(arrs, parts)]
            d //= 2
    score, _, x1, y1, x2, y2 = arrs

    def col_cat(a):
        return jnp.concatenate(
            [_colify(a[r:r + 1, :], eye) for r in range(_NBLK)], axis=0)

    x1c, y1c, x2c, y2c, sc_c = (col_cat(a) for a in (x1, y1, x2, y2, score))
    area_row = (x2 - x1) * (y2 - y1)
    area_col = (x2c - x1c) * (y2c - y1c)

    keepcol = (sc_c > 0.0).astype(f32)

    up_tri = (_iota2((128, 128), 1) > _iota2((128, 128), 0)).astype(f32)

    for bi in range(_NBLK):
        lo = bi * 128
        rx1, ry1, rx2, ry2 = (a[bi:bi + 1, :] for a in (x1, y1, x2, y2))
        cx1, cy1, cx2, cy2 = (c[lo:lo + 128] for c in (x1c, y1c, x2c, y2c))
        keep_row = _rowify(keepcol[lo:lo + 128], eye)

        xl = jnp.maximum(cx1, rx1)
        yt = jnp.maximum(cy1, ry1)
        xr = jnp.minimum(cx2, rx2)
        yb = jnp.minimum(cy2, ry2)
        inter = jnp.maximum(xr - xl, 0.0) * jnp.maximum(yb - yt, 0.0)
        iou = inter / (area_col[lo:lo + 128] + area_row[bi:bi + 1, :] - inter)
        s_mat = jnp.where(iou > _NMS_THRESH, up_tri, 0.0)

        init_row = keep_row

        def cond(c):
            y, yp = c
            return jnp.sum(jnp.abs(y - yp)) > 0.0

        def body(c):
            y, _ = c
            ycol = _colify(y, eye)
            sup = jnp.sum(s_mat * ycol, axis=0, keepdims=True)
            return init_row * jnp.where(sup > 0.0, 0.0, 1.0), y

        keep_row, _ = jax.lax.while_loop(cond, body,
                                         (init_row, init_row - 1.0))
        pieces = ([keepcol[:lo]] if lo else []) + [_colify(keep_row, eye)]

        if lo + 128 < _ACT:
            qx1 = x1c[lo + 128:]
            qy1 = y1c[lo + 128:]
            qx2 = x2c[lo + 128:]
            qy2 = y2c[lo + 128:]
            xl = jnp.maximum(qx1, rx1)
            yt = jnp.maximum(qy1, ry1)
            xr = jnp.minimum(qx2, rx2)
            yb = jnp.minimum(qy2, ry2)
            inter = jnp.maximum(xr - xl, 0.0) * jnp.maximum(yb - yt, 0.0)
            iou = inter / (area_col[lo + 128:] + area_row[bi:bi + 1, :] - inter)
            hit = jnp.where(iou > _NMS_THRESH, keep_row, 0.0)
            sup = jnp.sum(hit, axis=1, keepdims=True)
            pieces.append(keepcol[lo + 128:] * jnp.where(sup > 0.0, 0.0, 1.0))
        keepcol = jnp.concatenate(pieces, axis=0)

    keeprows = jnp.concatenate(
        [_rowify(keepcol[r * 128:(r + 1) * 128], eye) for r in range(_NBLK)],
        axis=0)
    triu = (_iota2((128, 128), 0) <= _iota2((128, 128), 1)).astype(f32)
    lane_cum = jnp.dot(keeprows, triu, preferred_element_type=f32,
                       precision=_HIGHEST)
    row_tot = lane_cum[:, 127:128]
    strict_lo = (_iota2((_NBLK, _NBLK), 0) > _iota2((_NBLK, _NBLK), 1)
                 ).astype(f32)
    row_ex = jnp.dot(strict_lo, row_tot, preferred_element_type=f32,
                     precision=_HIGHEST)
    posi = (lane_cum + row_ex - 1.0).astype(jnp.int32)

    out_iota = _iota2((_OUT_PAD, 128), 0)
    acc = jnp.zeros((_OUT_PAD, 8), f32)
    zeros3 = jnp.zeros((128, 3), f32)
    for bi in range(_NBLK):
        lo = bi * 128
        hot = jnp.where(out_iota == posi[bi:bi + 1, :],
                        keeprows[bi:bi + 1, :], 0.0)
        vals = jnp.concatenate(
            [x1c[lo:lo + 128], y1c[lo:lo + 128], x2c[lo:lo + 128],
             y2c[lo:lo + 128], sc_c[lo:lo + 128], zeros3], axis=1)
        acc = acc + jnp.dot(hot, vals, preferred_element_type=f32,
                            precision=_HIGHEST)
    out_ref[...] = acc


def _run_filter(logit, deltas, anchors, img_h, img_w):
    return pl.pallas_call(
        functools.partial(_filter_kernel, img_h=img_h, img_w=img_w),
        in_specs=[
            pl.BlockSpec((64, 128), lambda: (0, 0)),
            pl.BlockSpec((4, 64, 128), lambda: (0, 0, 0)),
            pl.BlockSpec((4, 64, 128), lambda: (0, 0, 0)),
        ],
        out_specs=pl.BlockSpec((_OUT_PAD, 8), lambda: (0, 0)),
        out_shape=jax.ShapeDtypeStruct((_OUT_PAD, 8), jnp.float32),
    )(logit, deltas, anchors)


def kernel(image, feat, rpn_w, rpn_b, cls_w, cls_b, rgs_w, rgs_b):
    f32 = jnp.float32
    xt = jnp.pad(jnp.transpose(feat[0], (1, 2, 0)), ((1, 1), (1, 1), (0, 0)))
    xcat = jnp.concatenate([xt[dy:dy + 24, dx:dx + 24, :].reshape(576, 512)
                            for dy in range(3) for dx in range(3)], axis=1)
    wcat = jnp.transpose(rpn_w, (2, 3, 1, 0)).reshape(4608, 512)
    cls_w2 = jnp.transpose(cls_w[:, :, 0, 0], (1, 0))
    rgs_w2 = jnp.transpose(rgs_w[:, :, 0, 0], (1, 0))

    cls_out, rgs_out = _run_trunk(
        xcat, wcat, cls_w2, rgs_w2, rpn_b.reshape(1, 512),
        cls_b.reshape(1, _K), rgs_b.reshape(1, 4 * _K))

    def to_grid(v):
        return jnp.pad(v, (0, _PAD_N - _N)).reshape(64, 128)

    logit = to_grid(cls_out.reshape(_N))
    d4 = rgs_out.reshape(_N, 4)
    deltas = jnp.stack([to_grid(d4[:, i]) for i in range(4)])
    anc = _anchors(image.shape, feat.shape)
    anchors = jnp.stack([to_grid(anc[:, i]) for i in range(4)])

    out = _run_filter(logit, deltas, anchors,
                      float(image.shape[-2]), float(image.shape[-1]))
    return out[:_TOPK, 0:4], out[:_TOPK, 4]
